# baseline (speedup 1.0000x reference)
import functools

import jax
import jax.numpy as jnp
from jax.experimental import pallas as pl

N = 1024
DIN = 64
DIM = 16
AH = 8
PH = 8
BLK_I = 64


def _proj_kernel(feature, pos, W1, b1, Wq, bq, Wk, bk, Wv, bv, Wp1, Wa1, ba1,
                 pp_o, qa_o, ka_o, v_o):
    f = jax.nn.relu(jnp.dot(feature[...], W1[...],
                            preferred_element_type=jnp.float32) + b1[...])
    q = jax.nn.relu(jnp.dot(f, Wq[...], preferred_element_type=jnp.float32) + bq[...])
    k = jax.nn.relu(jnp.dot(f, Wk[...], preferred_element_type=jnp.float32) + bk[...])
    v = jax.nn.relu(jnp.dot(f, Wv[...], preferred_element_type=jnp.float32) + bv[...])
    pp_o[...] = jnp.dot(pos[...], Wp1[...], preferred_element_type=jnp.float32)
    qa_o[...] = jnp.dot(q, Wa1[...], preferred_element_type=jnp.float32) + ba1[...]
    ka_o[...] = jnp.dot(k, Wa1[...], preferred_element_type=jnp.float32)
    v_o[...] = v


def _attn_kernel(pp, qa, ka, v, bp1, Wp2, bp2, Wa1, Wa2, ba2, W2, b2, out):
    i0 = pl.program_id(0) * BLK_I
    ppi = pp[pl.ds(i0, BLK_I), :]
    kai = ka[pl.ds(i0, BLK_I), :]
    ppj = pp[...]
    qaj = qa[...]

    a = jax.nn.relu(ppj[None, :, :] - ppi[:, None, :] + bp1[...][None, :, :])
    a2 = a.reshape(BLK_I * N, PH)
    pe = jax.nn.relu(jnp.dot(a2, Wp2[...], preferred_element_type=jnp.float32)
                     + bp2[...])
    qk = (qaj[None, :, :] - kai[:, None, :]).reshape(BLK_I * N, AH)
    u = jax.nn.relu(jnp.dot(pe, Wa1[...], preferred_element_type=jnp.float32) + qk)
    e = jax.nn.relu(jnp.dot(u, Wa2[...], preferred_element_type=jnp.float32)
                    + ba2[...])
    e = e.reshape(BLK_I, N, DIM)
    m = jnp.max(e, axis=1, keepdims=True)
    w = jnp.exp(e - m)
    s = jnp.sum(w, axis=1)
    o = jnp.sum(w * v[...][None, :, :], axis=1) / s
    out[...] = jnp.dot(o, W2[...], preferred_element_type=jnp.float32) + b2[...]


@functools.partial(jax.jit, static_argnames=())
def kernel(feature, pos, W1, b1, Wq, bq, Wk, bk, Wv, bv,
           Wp1, bp1, Wp2, bp2, Wa1, ba1, Wa2, ba2, W2, b2):
    feat2 = feature.reshape(N, DIN)
    pos2 = pos.reshape(N, 3)
    r = lambda x: x.reshape(1, -1)

    pp, qa, ka, v = pl.pallas_call(
        _proj_kernel,
        out_shape=(
            jax.ShapeDtypeStruct((N, PH), jnp.float32),
            jax.ShapeDtypeStruct((N, AH), jnp.float32),
            jax.ShapeDtypeStruct((N, AH), jnp.float32),
            jax.ShapeDtypeStruct((N, DIM), jnp.float32),
        ),
    )(feat2, pos2, W1, r(b1), Wq, r(bq), Wk, r(bk), Wv, r(bv), Wp1, Wa1, r(ba1))

    grid = (N // BLK_I,)
    full = lambda shape: pl.BlockSpec(shape, lambda i: tuple(0 for _ in shape))
    out = pl.pallas_call(
        _attn_kernel,
        grid=grid,
        in_specs=[
            full((N, PH)), full((N, AH)), full((N, AH)), full((N, DIM)),
            full((1, PH)), full((PH, DIM)), full((1, DIM)),
            full((DIM, AH)), full((AH, DIM)), full((1, DIM)),
            full((DIM, DIM)), full((1, DIM)),
        ],
        out_specs=pl.BlockSpec((BLK_I, DIM), lambda i: (i, 0)),
        out_shape=jax.ShapeDtypeStruct((N, DIM), jnp.float32),
    )(pp, qa, ka, v, r(bp1), Wp2, r(bp2), Wa1, Wa2, r(ba2), W2, r(b2))

    return out.reshape(1, N, DIM)


# speedup vs baseline: 7.7111x; 7.7111x over previous
import jax
import jax.numpy as jnp
from jax.experimental import pallas as pl

N = 1024
DIN = 64
DIM = 16
AH = 8
PH = 8
BLK_I = 64


def _proj_kernel(featT, posT, W1, b1, Wq, bq, Wk, bk, Wv, bv, Wp1, Wa1, ba1,
                 ppT_o, qaT_o, kaT_o, vT_o, ppr_o, kar_o):
    fT = jax.nn.relu(jnp.dot(W1[...], featT[...],
                             preferred_element_type=jnp.float32) + b1[...])
    qT = jax.nn.relu(jnp.dot(Wq[...], fT, preferred_element_type=jnp.float32) + bq[...])
    kT = jax.nn.relu(jnp.dot(Wk[...], fT, preferred_element_type=jnp.float32) + bk[...])
    vT_o[...] = jax.nn.relu(jnp.dot(Wv[...], fT, preferred_element_type=jnp.float32)
                            + bv[...])
    ppT = jnp.dot(Wp1[...], posT[...], preferred_element_type=jnp.float32)
    kaT = jnp.dot(Wa1[...], kT, preferred_element_type=jnp.float32)
    ppT_o[...] = ppT
    qaT_o[...] = jnp.dot(Wa1[...], qT, preferred_element_type=jnp.float32) + ba1[...]
    kaT_o[...] = kaT
    ppr_o[...] = ppT.T
    kar_o[...] = kaT.T


def _attn_kernel(ppT, qaT, vT, ppr, kar, bp1, Wp2T, bp2, Wa1T, Wa2T, ba2,
                 W2, b2, out):
    i0 = pl.program_id(0) * BLK_I
    ppi = ppr[pl.ds(i0, BLK_I), :][:, :, None]
    kai = kar[pl.ds(i0, BLK_I), :][:, :, None]
    ppj = ppT[...][None, :, :]
    qaj = qaT[...][None, :, :]

    def bdot(w, x, co):
        wb = jnp.broadcast_to(w[None, :, :], (BLK_I,) + w.shape)
        return jax.lax.dot_general(
            wb, x, (((2,), (1,)), ((0,), (0,))),
            preferred_element_type=jnp.float32)

    a = jax.nn.relu(ppj - ppi + bp1[...][None, :, :])
    pe = jax.nn.relu(bdot(Wp2T[...], a, DIM) + bp2[...][None, :, :])
    u = jax.nn.relu(bdot(Wa1T[...], pe, AH) + qaj - kai)
    e = jax.nn.relu(bdot(Wa2T[...], u, DIM) + ba2[...][None, :, :])
    m = jnp.max(e, axis=2, keepdims=True)
    w = jnp.exp(e - m)
    s = jnp.sum(w, axis=2, keepdims=True)
    o = jnp.sum(w * vT[...][None, :, :], axis=2, keepdims=True) / s
    o = o.reshape(BLK_I, DIM)
    out[...] = jnp.dot(o, W2[...], preferred_element_type=jnp.float32) + b2[...]


def kernel(feature, pos, W1, b1, Wq, bq, Wk, bk, Wv, bv,
           Wp1, bp1, Wp2, bp2, Wa1, ba1, Wa2, ba2, W2, b2):
    featT = feature.reshape(N, DIN).T
    posT = pos.reshape(N, 3).T
    c = lambda x: x.reshape(-1, 1)

    ppT, qaT, kaT, vT, ppr, kar = pl.pallas_call(
        _proj_kernel,
        out_shape=(
            jax.ShapeDtypeStruct((PH, N), jnp.float32),
            jax.ShapeDtypeStruct((AH, N), jnp.float32),
            jax.ShapeDtypeStruct((AH, N), jnp.float32),
            jax.ShapeDtypeStruct((DIM, N), jnp.float32),
            jax.ShapeDtypeStruct((N, PH), jnp.float32),
            jax.ShapeDtypeStruct((N, AH), jnp.float32),
        ),
    )(featT, posT, W1.T, c(b1), Wq.T, c(bq), Wk.T, c(bk), Wv.T, c(bv),
      Wp1.T, Wa1.T, c(ba1))
    del kaT

    grid = (N // BLK_I,)
    full = lambda shape: pl.BlockSpec(shape, lambda i: tuple(0 for _ in shape))
    out = pl.pallas_call(
        _attn_kernel,
        grid=grid,
        in_specs=[
            full((PH, N)), full((AH, N)), full((DIM, N)),
            full((N, PH)), full((N, AH)),
            full((PH, 1)), full((DIM, PH)), full((DIM, 1)),
            full((AH, DIM)), full((DIM, AH)), full((DIM, 1)),
            full((DIM, DIM)), full((1, DIM)),
        ],
        out_specs=pl.BlockSpec((BLK_I, DIM), lambda i: (i, 0)),
        out_shape=jax.ShapeDtypeStruct((N, DIM), jnp.float32),
    )(ppT, qaT, vT, ppr, kar, c(bp1), Wp2.T, c(bp2), Wa1.T, Wa2.T, c(ba2),
      W2, b2.reshape(1, DIM))

    return out.reshape(1, N, DIM)


# speedup vs baseline: 7.9253x; 1.0278x over previous
import jax
import jax.numpy as jnp
from jax.experimental import pallas as pl

N = 1024
DIN = 64
DIM = 16
AH = 8
PH = 8
BLK_I = 128


def _proj_kernel(featT, posT, W1, b1, Wq, bq, Wk, bk, Wv, bv, Wp1, Wa1, ba1,
                 ppT_o, qaT_o, kaT_o, vT_o, ppr_o, kar_o):
    fT = jax.nn.relu(jnp.dot(W1[...], featT[...],
                             preferred_element_type=jnp.float32) + b1[...])
    qT = jax.nn.relu(jnp.dot(Wq[...], fT, preferred_element_type=jnp.float32) + bq[...])
    kT = jax.nn.relu(jnp.dot(Wk[...], fT, preferred_element_type=jnp.float32) + bk[...])
    vT_o[...] = jax.nn.relu(jnp.dot(Wv[...], fT, preferred_element_type=jnp.float32)
                            + bv[...])
    ppT = jnp.dot(Wp1[...], posT[...], preferred_element_type=jnp.float32)
    kaT = jnp.dot(Wa1[...], kT, preferred_element_type=jnp.float32)
    ppT_o[...] = ppT
    qaT_o[...] = jnp.dot(Wa1[...], qT, preferred_element_type=jnp.float32) + ba1[...]
    kaT_o[...] = kaT
    ppr_o[...] = ppT.T
    kar_o[...] = kaT.T


def _attn_kernel(ppT, qaT, vT, ppr, kar, bp1, Wp2T, bp2, Wa1T, Wa2T, ba2,
                 W2, b2, out):
    i0 = pl.program_id(0) * BLK_I
    ppi = ppr[pl.ds(i0, BLK_I), :][:, :, None]
    kai = kar[pl.ds(i0, BLK_I), :][:, :, None]
    ppj = ppT[...][None, :, :]
    qaj = qaT[...][None, :, :]

    def bdot(w, x, co):
        wb = jnp.broadcast_to(w[None, :, :], (BLK_I,) + w.shape)
        return jax.lax.dot_general(
            wb, x, (((2,), (1,)), ((0,), (0,))),
            preferred_element_type=jnp.float32)

    a = jax.nn.relu(ppj - ppi + bp1[...][None, :, :])
    pe = jax.nn.relu(bdot(Wp2T[...], a, DIM) + bp2[...][None, :, :])
    u = jax.nn.relu(bdot(Wa1T[...], pe, AH) + qaj - kai)
    e = jax.nn.relu(bdot(Wa2T[...], u, DIM) + ba2[...][None, :, :])
    m = jnp.max(e, axis=2, keepdims=True)
    w = jnp.exp(e - m)
    s = jnp.sum(w, axis=2, keepdims=True)
    o = jnp.sum(w * vT[...][None, :, :], axis=2, keepdims=True) / s
    o = o.reshape(BLK_I, DIM)
    out[...] = jnp.dot(o, W2[...], preferred_element_type=jnp.float32) + b2[...]


def kernel(feature, pos, W1, b1, Wq, bq, Wk, bk, Wv, bv,
           Wp1, bp1, Wp2, bp2, Wa1, ba1, Wa2, ba2, W2, b2):
    featT = feature.reshape(N, DIN).T
    posT = pos.reshape(N, 3).T
    c = lambda x: x.reshape(-1, 1)

    ppT, qaT, kaT, vT, ppr, kar = pl.pallas_call(
        _proj_kernel,
        out_shape=(
            jax.ShapeDtypeStruct((PH, N), jnp.float32),
            jax.ShapeDtypeStruct((AH, N), jnp.float32),
            jax.ShapeDtypeStruct((AH, N), jnp.float32),
            jax.ShapeDtypeStruct((DIM, N), jnp.float32),
            jax.ShapeDtypeStruct((N, PH), jnp.float32),
            jax.ShapeDtypeStruct((N, AH), jnp.float32),
        ),
    )(featT, posT, W1.T, c(b1), Wq.T, c(bq), Wk.T, c(bk), Wv.T, c(bv),
      Wp1.T, Wa1.T, c(ba1))
    del kaT

    grid = (N // BLK_I,)
    full = lambda shape: pl.BlockSpec(shape, lambda i: tuple(0 for _ in shape))
    out = pl.pallas_call(
        _attn_kernel,
        grid=grid,
        in_specs=[
            full((PH, N)), full((AH, N)), full((DIM, N)),
            full((N, PH)), full((N, AH)),
            full((PH, 1)), full((DIM, PH)), full((DIM, 1)),
            full((AH, DIM)), full((DIM, AH)), full((DIM, 1)),
            full((DIM, DIM)), full((1, DIM)),
        ],
        out_specs=pl.BlockSpec((BLK_I, DIM), lambda i: (i, 0)),
        out_shape=jax.ShapeDtypeStruct((N, DIM), jnp.float32),
    )(ppT, qaT, vT, ppr, kar, c(bp1), Wp2.T, c(bp2), Wa1.T, Wa2.T, c(ba2),
      W2, b2.reshape(1, DIM))

    return out.reshape(1, N, DIM)
